# initial kernel scaffold (unmeasured)
import jax
import jax.numpy as jnp
from jax import lax
from jax.experimental import pallas as pl
from jax.experimental.pallas import tpu as pltpu

N_DEV = 16
M = 4096
N = 8192
CHUNK = M // N_DEV


def _allreduce_body(
    p_ref,
    q_ref,
    amax_ref,
    comm,
    send,
    local,
    qbuf,
    amax_all,
    send_sem, recv_sem,
    amax_send_sems, amax_recv_sems,
    local_sem, out_sem,
    credit_sem,
):
    d = lax.axis_index("i")
    left = lax.rem(d - 1 + N_DEV, N_DEV)
    right = lax.rem(d + 1, N_DEV)

    barrier_sem = pltpu.get_barrier_semaphore()
    pl.semaphore_signal(barrier_sem, inc=1, device_id=(left,),
                        device_id_type=pl.DeviceIdType.MESH)
    pl.semaphore_signal(barrier_sem, inc=1, device_id=(right,),
                        device_id_type=pl.DeviceIdType.MESH)
    pl.semaphore_wait(barrier_sem, 2)

    cp = pltpu.make_async_copy(p_ref.at[pl.ds(d * CHUNK, CHUNK)], send, local_sem)
    cp.start()
    cp.wait()

    for s in range(N_DEV - 1):
        c_recv = lax.rem(d - s - 1 + 2 * N_DEV, N_DEV)
        if s >= 1:
            pl.semaphore_wait(credit_sem, 1)
        rdma = pltpu.make_async_remote_copy(
            src_ref=send, dst_ref=comm,
            send_sem=send_sem, recv_sem=recv_sem,
            device_id=(right,), device_id_type=pl.DeviceIdType.MESH,
        )
        rdma.start()
        cp = pltpu.make_async_copy(
            p_ref.at[pl.ds(c_recv * CHUNK, CHUNK)], local, local_sem)
        cp.start()
        rdma.wait_send()
        rdma.wait_recv()
        cp.wait()
        send[...] = comm[...] + local[...]
        if s < N_DEV - 2:
            pl.semaphore_signal(credit_sem, inc=1, device_id=(left,),
                                device_id_type=pl.DeviceIdType.MESH)

    y = jnp.maximum(send[...], 0.0)
    m_local = jnp.max(y)
    amax_all[pl.ds(d, 1)] = jnp.full((1, 128), m_local, jnp.float32)
    for o in range(1, N_DEV):
        tgt = lax.rem(d + o, N_DEV)
        rdma = pltpu.make_async_remote_copy(
            src_ref=amax_all.at[pl.ds(d, 1)],
            dst_ref=amax_all.at[pl.ds(d, 1)],
            send_sem=amax_send_sems.at[o],
            recv_sem=amax_recv_sems.at[d],
            device_id=(tgt,), device_id_type=pl.DeviceIdType.MESH,
        )
        rdma.start()
    for o in range(1, N_DEV):
        src = lax.rem(d - o + N_DEV, N_DEV)
        rdma = pltpu.make_async_remote_copy(
            src_ref=amax_all.at[pl.ds(src, 1)],
            dst_ref=amax_all.at[pl.ds(src, 1)],
            send_sem=amax_send_sems.at[o],
            recv_sem=amax_recv_sems.at[src],
            device_id=(src,), device_id_type=pl.DeviceIdType.MESH,
        )
        rdma.wait_recv()
    for o in range(1, N_DEV):
        rdma = pltpu.make_async_remote_copy(
            src_ref=amax_all.at[pl.ds(d, 1)],
            dst_ref=amax_all.at[pl.ds(d, 1)],
            send_sem=amax_send_sems.at[o],
            recv_sem=amax_recv_sems.at[d],
            device_id=(right,), device_id_type=pl.DeviceIdType.MESH,
        )
        rdma.wait_send()

    g_amax = jnp.max(amax_all[...])
    amax_ref[...] = jnp.full((8, 128), g_amax, jnp.float32)

    scale = g_amax / 448.0
    own = lax.rem(d + 1, N_DEV)
    qbuf[0] = (y / scale).astype(jnp.float8_e4m3fn)
    cp = pltpu.make_async_copy(
        qbuf.at[0], q_ref.at[pl.ds(own * CHUNK, CHUNK)], out_sem)
    cp.start()
    cp.wait()

    for t in range(N_DEV - 1):
        send_slot = t % 2
        recv_slot = (t + 1) % 2
        if t >= 1:
            pl.semaphore_wait(credit_sem, 1)
        rdma = pltpu.make_async_remote_copy(
            src_ref=qbuf.at[send_slot], dst_ref=qbuf.at[recv_slot],
            send_sem=send_sem, recv_sem=recv_sem,
            device_id=(right,), device_id_type=pl.DeviceIdType.MESH,
        )
        rdma.start()
        rdma.wait_send()
        rdma.wait_recv()
        c = lax.rem(d - t + N_DEV, N_DEV)
        cp = pltpu.make_async_copy(
            qbuf.at[recv_slot], q_ref.at[pl.ds(c * CHUNK, CHUNK)], out_sem)
        cp.start()
        cp.wait()
        if t < N_DEV - 2:
            pl.semaphore_signal(credit_sem, inc=1, device_id=(left,),
                                device_id_type=pl.DeviceIdType.MESH)


def _allreduce_quant(partial):
    q, amax = pl.pallas_call(
        _allreduce_body,
        out_shape=[
            jax.ShapeDtypeStruct((M, N), jnp.float8_e4m3fn),
            jax.ShapeDtypeStruct((8, 128), jnp.float32),
        ],
        in_specs=[pl.BlockSpec(memory_space=pltpu.ANY)],
        out_specs=[
            pl.BlockSpec(memory_space=pltpu.ANY),
            pl.BlockSpec(memory_space=pltpu.VMEM),
        ],
        scratch_shapes=[
            pltpu.VMEM((CHUNK, N), jnp.float32),
            pltpu.VMEM((CHUNK, N), jnp.float32),
            pltpu.VMEM((CHUNK, N), jnp.float32),
            pltpu.VMEM((2, CHUNK, N), jnp.float8_e4m3fn),
            pltpu.VMEM((N_DEV, 128), jnp.float32),
            pltpu.SemaphoreType.DMA,
            pltpu.SemaphoreType.DMA,
            pltpu.SemaphoreType.DMA((N_DEV,)),
            pltpu.SemaphoreType.DMA((N_DEV,)),
            pltpu.SemaphoreType.DMA,
            pltpu.SemaphoreType.DMA,
            pltpu.SemaphoreType.REGULAR,
        ],
        compiler_params=pltpu.CompilerParams(collective_id=0),
    )(partial)
    return q, amax


def kernel(x, w_mat):
    partial = jnp.dot(
        x.astype(jnp.bfloat16), w_mat.astype(jnp.bfloat16),
        preferred_element_type=jnp.float32,
    )
    q, amax = _allreduce_quant(partial)
    scale = amax[0, 0] / 448.0
    return q.astype(jnp.float32) * scale


# baseline (device time: 1989710 ns/iter reference)
import jax
import jax.numpy as jnp
from jax import lax
from jax.experimental import pallas as pl
from jax.experimental.pallas import tpu as pltpu

N_DEV = 16
M = 4096
N = 8192
CHUNK = M // N_DEV


def _allreduce_body(
    p_ref,
    q_ref,
    amax_ref,
    comm,
    send,
    local,
    qbuf,
    amax_all,
    send_sem, recv_sem,
    amax_send_sems, amax_recv_sems,
    local_sem, out_sem,
    credit_sem,
):
    d = lax.axis_index("i")
    left = lax.rem(d - 1 + N_DEV, N_DEV)
    right = lax.rem(d + 1, N_DEV)

    barrier_sem = pltpu.get_barrier_semaphore()
    pl.semaphore_signal(barrier_sem, inc=1, device_id=(left,),
                        device_id_type=pl.DeviceIdType.MESH)
    pl.semaphore_signal(barrier_sem, inc=1, device_id=(right,),
                        device_id_type=pl.DeviceIdType.MESH)
    pl.semaphore_wait(barrier_sem, 2)

    cp = pltpu.make_async_copy(p_ref.at[pl.ds(d * CHUNK, CHUNK)], send, local_sem)
    cp.start()
    cp.wait()

    for s in range(N_DEV - 1):
        c_recv = lax.rem(d - s - 1 + 2 * N_DEV, N_DEV)
        if s >= 1:
            pl.semaphore_wait(credit_sem, 1)
        rdma = pltpu.make_async_remote_copy(
            src_ref=send, dst_ref=comm,
            send_sem=send_sem, recv_sem=recv_sem,
            device_id=(right,), device_id_type=pl.DeviceIdType.MESH,
        )
        rdma.start()
        cp = pltpu.make_async_copy(
            p_ref.at[pl.ds(c_recv * CHUNK, CHUNK)], local, local_sem)
        cp.start()
        rdma.wait_send()
        rdma.wait_recv()
        cp.wait()
        send[...] = comm[...] + local[...]
        if s < N_DEV - 2:
            pl.semaphore_signal(credit_sem, inc=1, device_id=(left,),
                                device_id_type=pl.DeviceIdType.MESH)

    y = jnp.maximum(send[...], 0.0)
    m_local = jnp.max(y)
    amax_all[pl.ds(d, 1)] = jnp.full((1, 128), m_local, jnp.float32)
    for o in range(1, N_DEV):
        tgt = lax.rem(d + o, N_DEV)
        rdma = pltpu.make_async_remote_copy(
            src_ref=amax_all.at[pl.ds(d, 1)],
            dst_ref=amax_all.at[pl.ds(d, 1)],
            send_sem=amax_send_sems.at[o],
            recv_sem=amax_recv_sems.at[d],
            device_id=(tgt,), device_id_type=pl.DeviceIdType.MESH,
        )
        rdma.start()
    for o in range(1, N_DEV):
        src = lax.rem(d - o + N_DEV, N_DEV)
        rdma = pltpu.make_async_remote_copy(
            src_ref=amax_all.at[pl.ds(src, 1)],
            dst_ref=amax_all.at[pl.ds(src, 1)],
            send_sem=amax_send_sems.at[o],
            recv_sem=amax_recv_sems.at[src],
            device_id=(src,), device_id_type=pl.DeviceIdType.MESH,
        )
        rdma.wait_recv()
    for o in range(1, N_DEV):
        rdma = pltpu.make_async_remote_copy(
            src_ref=amax_all.at[pl.ds(d, 1)],
            dst_ref=amax_all.at[pl.ds(d, 1)],
            send_sem=amax_send_sems.at[o],
            recv_sem=amax_recv_sems.at[d],
            device_id=(right,), device_id_type=pl.DeviceIdType.MESH,
        )
        rdma.wait_send()

    g_amax = jnp.max(amax_all[...])
    amax_ref[...] = jnp.full((8, 128), g_amax, jnp.float32)

    scale = g_amax / 448.0
    own = lax.rem(d + 1, N_DEV)
    qbuf[0] = (y / scale).astype(jnp.float8_e4m3fn)
    cp = pltpu.make_async_copy(
        qbuf.at[0], q_ref.at[pl.ds(own * CHUNK, CHUNK)], out_sem)
    cp.start()
    cp.wait()

    for t in range(N_DEV - 1):
        send_slot = t % 2
        recv_slot = (t + 1) % 2
        if t >= 1:
            pl.semaphore_wait(credit_sem, 1)
        rdma = pltpu.make_async_remote_copy(
            src_ref=qbuf.at[send_slot], dst_ref=qbuf.at[recv_slot],
            send_sem=send_sem, recv_sem=recv_sem,
            device_id=(right,), device_id_type=pl.DeviceIdType.MESH,
        )
        rdma.start()
        rdma.wait_send()
        rdma.wait_recv()
        c = lax.rem(d - t + N_DEV, N_DEV)
        cp = pltpu.make_async_copy(
            qbuf.at[recv_slot], q_ref.at[pl.ds(c * CHUNK, CHUNK)], out_sem)
        cp.start()
        cp.wait()
        if t < N_DEV - 2:
            pl.semaphore_signal(credit_sem, inc=1, device_id=(left,),
                                device_id_type=pl.DeviceIdType.MESH)


def _allreduce_quant(partial):
    q, amax = pl.pallas_call(
        _allreduce_body,
        out_shape=[
            jax.ShapeDtypeStruct((M, N), jnp.float8_e4m3fn),
            jax.ShapeDtypeStruct((8, 128), jnp.float32),
        ],
        in_specs=[pl.BlockSpec(memory_space=pl.ANY)],
        out_specs=[
            pl.BlockSpec(memory_space=pl.ANY),
            pl.BlockSpec(memory_space=pltpu.VMEM),
        ],
        scratch_shapes=[
            pltpu.VMEM((CHUNK, N), jnp.float32),
            pltpu.VMEM((CHUNK, N), jnp.float32),
            pltpu.VMEM((CHUNK, N), jnp.float32),
            pltpu.VMEM((2, CHUNK, N), jnp.float8_e4m3fn),
            pltpu.VMEM((N_DEV, 128), jnp.float32),
            pltpu.SemaphoreType.DMA,
            pltpu.SemaphoreType.DMA,
            pltpu.SemaphoreType.DMA((N_DEV,)),
            pltpu.SemaphoreType.DMA((N_DEV,)),
            pltpu.SemaphoreType.DMA,
            pltpu.SemaphoreType.DMA,
            pltpu.SemaphoreType.REGULAR,
        ],
        compiler_params=pltpu.CompilerParams(
            collective_id=0, vmem_limit_bytes=100 * 1024 * 1024
        ),
    )(partial)
    return q, amax


def kernel(x, w_mat):
    partial = jnp.dot(
        x.astype(jnp.bfloat16), w_mat.astype(jnp.bfloat16),
        preferred_element_type=jnp.float32,
    )
    q, amax = _allreduce_quant(partial)
    scale = amax[0, 0] / 448.0
    return q.astype(jnp.float32) * scale
